# baseline (speedup 1.0000x reference)
import functools

import jax
import jax.numpy as jnp
from jax import lax
from jax.experimental import pallas as pl
from jax.experimental.pallas import tpu as pltpu
from jax.experimental.pallas import tpu_sc as plsc

NN = 10000
EE = 320000
DD = 128
HH = 16
BB = 64

NC, NS = 2, 16
NT = NC * NS
CHW = 128
KG = 16
NG = 5
CH = KG * NG
EPT = CH * CHW
EPAD = NT * EPT
NPAD = 10240
ROWS_PT = NPAD // NS
YB = NN * HH // 128
PB = NPAD * HH // 128


def _mm1_body(x8_ref, w1_ref, o_ref):
    for g in range(8):
        xg = x8_ref[:, DD * g:DD * (g + 1)]
        o_ref[:, HH * g:HH * (g + 1)] = jnp.dot(
            xg, w1_ref[...], preferred_element_type=jnp.float32)


def _final_body(p_ref, y_ref, b1_ref, w2_ref, b2_ref, bat8_ref, wc_ref,
                bc_ref, o_ref):
    seg = lax.broadcasted_iota(jnp.int32, (BB, YB), 0)
    sums = jnp.zeros((BB, HH), jnp.float32)
    counts = jnp.zeros((BB, 1), jnp.float32)
    for g in range(8):
        sl = slice(HH * g, HH * (g + 1))
        z_g = jnp.maximum(
            p_ref[0, :YB, sl] + p_ref[1, :YB, sl] + y_ref[:, sl] + b1_ref[...], 0.0)
        h2_g = jnp.maximum(
            jnp.dot(z_g, w2_ref[...], preferred_element_type=jnp.float32)
            + b2_ref[...], 0.0)
        m_g = (seg == bat8_ref[g]).astype(jnp.float32)
        sums = sums + jnp.dot(m_g, h2_g, preferred_element_type=jnp.float32)
        counts = counts + jnp.sum(m_g, axis=1, keepdims=True)
    pooled = sums / jnp.maximum(counts, 1.0)
    o_ref[...] = jnp.dot(pooled, wc_ref[...], preferred_element_type=jnp.float32) + bc_ref[...]


_SC_MESH = plsc.VectorSubcoreMesh(core_axis_name="c", subcore_axis_name="s")


@functools.partial(
    pl.kernel,
    mesh=_SC_MESH,
    compiler_params=pltpu.CompilerParams(use_tc_tiling_on_sc=False),
    out_type=jax.ShapeDtypeStruct((NC, NPAD, HH), jnp.float32),
    scratch_types=[
        pltpu.VMEM((2, CH, CHW), jnp.int32),
        pltpu.VMEM((2, KG, CHW, HH), jnp.float32),
        pltpu.VMEM_SHARED((NPAD, HH), jnp.float32),
        pltpu.VMEM_SHARED((NPAD, HH), jnp.float32),
        pltpu.SemaphoreType.DMA,
        pltpu.SemaphoreType.DMA,
    ],
)
def _sc_agg(y_hbm, src_hbm, dst_hbm, out_hbm, idx_v, rows_v, acc_sh, y_sh,
            sem_g, sem_s):
    c = lax.axis_index("c")
    s = lax.axis_index("s")
    wid = s * NC + c
    pltpu.sync_copy(src_hbm.at[wid], idx_v.at[0])
    pltpu.sync_copy(dst_hbm.at[wid], idx_v.at[1])
    for r in range(CHW):
        rows_v[0, 0, r, :] = jnp.zeros((HH,), jnp.float32)
    for k in range(ROWS_PT // CHW):
        pltpu.sync_copy(rows_v.at[0, 0], acc_sh.at[pl.ds(s * ROWS_PT + k * CHW, CHW)])
    yrow0 = s * (NN // NS)
    pltpu.sync_copy(y_hbm.at[pl.ds(yrow0, NN // NS)], y_sh.at[pl.ds(yrow0, NN // NS)])
    plsc.subcore_barrier()

    def fire_gathers(g, b):
        for i in range(KG):
            pltpu.async_copy(y_sh.at[idx_v.at[0, g * KG + i]], rows_v.at[b, i],
                             sem_g)

    def wait_gathers(b):
        for i in range(KG):
            pltpu.make_async_copy(y_sh.at[idx_v.at[0, i]], rows_v.at[b, i],
                                  sem_g).wait()

    def fire_scatters(g, b):
        for i in range(KG):
            pltpu.async_copy(rows_v.at[b, i], acc_sh.at[idx_v.at[1, g * KG + i]],
                             sem_s, add=True)

    def wait_scatters(b):
        for i in range(KG):
            pltpu.make_async_copy(rows_v.at[b, i], acc_sh.at[idx_v.at[1, i]],
                                  sem_s).wait()

    fire_gathers(0, 0)

    def body(g, carry):
        b = lax.rem(g, 2)
        wait_gathers(b)
        fire_scatters(g, b)

        @pl.when(g >= 1)
        def _drain_prev():
            wait_scatters(1 - b)

        @pl.when(g + 1 < NG)
        def _fire_next():
            fire_gathers(g + 1, 1 - b)

        return carry

    lax.fori_loop(0, NG, body, 0)
    wait_scatters((NG - 1) % 2)
    plsc.subcore_barrier()
    row0 = s * ROWS_PT
    pltpu.sync_copy(acc_sh.at[pl.ds(row0, ROWS_PT)],
                    out_hbm.at[c, pl.ds(row0, ROWS_PT)])


def kernel(x, edge_index, batch, W1, b1, W2, b2, Wc, bc):
    y_blk = pl.pallas_call(
        _mm1_body,
        out_shape=jax.ShapeDtypeStruct((YB, 128), jnp.float32),
    )(x.reshape(YB, 8 * DD), W1)
    y = y_blk.reshape(NN, HH)

    src_t = jnp.pad(edge_index[0], (0, EPAD - EE),
                    constant_values=NN).reshape(NT, CH, CHW)
    dst_t = jnp.pad(edge_index[1], (0, EPAD - EE),
                    constant_values=NN).reshape(NT, CH, CHW)

    partials = _sc_agg(y, src_t, dst_t)
    p_blk = partials.reshape(NC, PB, 128)

    out = pl.pallas_call(
        _final_body,
        out_shape=jax.ShapeDtypeStruct((BB, 2), jnp.float32),
    )(p_blk, y_blk, b1.reshape(1, HH), W2, b2.reshape(1, HH),
      batch.reshape(YB, 8).T, Wc, bc.reshape(1, 2))
    return out


# speedup vs baseline: 1.0230x; 1.0230x over previous
import functools

import jax
import jax.numpy as jnp
from jax import lax
from jax.experimental import pallas as pl
from jax.experimental.pallas import tpu as pltpu
from jax.experimental.pallas import tpu_sc as plsc

NN = 10000
EE = 320000
DD = 128
HH = 16
BB = 64

NC, NS = 2, 16
NT = NC * NS
CHW = 128
KG = 16
NG = 5
CH = KG * NG
EPT = CH * CHW
EPAD = NT * EPT
NPAD = 10240
ROWS_PT = NPAD // NS
YB = NN * HH // 128
PB = NPAD * HH // 128


def _mm1_body(x8_ref, w1_ref, o_ref):
    for g in range(8):
        xg = x8_ref[:, DD * g:DD * (g + 1)]
        o_ref[:, HH * g:HH * (g + 1)] = jnp.dot(
            xg, w1_ref[...], preferred_element_type=jnp.float32)


def _final_body(p_ref, y_ref, b1_ref, w2_ref, b2_ref, bat8_ref, wc_ref,
                bc_ref, o_ref):
    seg = lax.broadcasted_iota(jnp.int32, (BB, YB), 0)
    sums = jnp.zeros((BB, HH), jnp.float32)
    counts = jnp.zeros((BB, 1), jnp.float32)
    for g in range(8):
        sl = slice(HH * g, HH * (g + 1))
        z_g = jnp.maximum(
            p_ref[0, :YB, sl] + p_ref[1, :YB, sl] + y_ref[:, sl] + b1_ref[...], 0.0)
        h2_g = jnp.maximum(
            jnp.dot(z_g, w2_ref[...], preferred_element_type=jnp.float32)
            + b2_ref[...], 0.0)
        m_g = (seg == bat8_ref[g]).astype(jnp.float32)
        sums = sums + jnp.dot(m_g, h2_g, preferred_element_type=jnp.float32)
        counts = counts + jnp.sum(m_g, axis=1, keepdims=True)
    pooled = sums / jnp.maximum(counts, 1.0)
    o_ref[...] = jnp.dot(pooled, wc_ref[...], preferred_element_type=jnp.float32) + bc_ref[...]


_SC_MESH = plsc.VectorSubcoreMesh(core_axis_name="c", subcore_axis_name="s")


@functools.partial(
    pl.kernel,
    mesh=_SC_MESH,
    compiler_params=pltpu.CompilerParams(use_tc_tiling_on_sc=False),
    out_type=jax.ShapeDtypeStruct((NC, NPAD, HH), jnp.float32),
    scratch_types=[
        pltpu.VMEM((2, CH, CHW), jnp.int32),
        pltpu.VMEM((KG, CHW, HH), jnp.float32),
        pltpu.VMEM_SHARED((NPAD, HH), jnp.float32),
        pltpu.VMEM_SHARED((NPAD, HH), jnp.float32),
        pltpu.SemaphoreType.DMA,
        pltpu.SemaphoreType.DMA,
    ],
)
def _sc_agg(y_hbm, src_hbm, dst_hbm, out_hbm, idx_v, rows_v, acc_sh, y_sh,
            sem_g, sem_s):
    c = lax.axis_index("c")
    s = lax.axis_index("s")
    wid = s * NC + c
    pltpu.sync_copy(src_hbm.at[wid], idx_v.at[0])
    pltpu.sync_copy(dst_hbm.at[wid], idx_v.at[1])
    for r in range(CHW):
        rows_v[0, r, :] = jnp.zeros((HH,), jnp.float32)
    for k in range(ROWS_PT // CHW):
        pltpu.sync_copy(rows_v.at[0], acc_sh.at[pl.ds(s * ROWS_PT + k * CHW, CHW)])
    yrow0 = s * (NN // NS)
    pltpu.sync_copy(y_hbm.at[pl.ds(yrow0, NN // NS)], y_sh.at[pl.ds(yrow0, NN // NS)])
    plsc.subcore_barrier()

    def body(g, carry):
        j0 = g * KG
        gathers = [
            pltpu.async_copy(y_sh.at[idx_v.at[0, j0 + i]], rows_v.at[i], sem_g)
            for i in range(KG)
        ]
        scatters = []
        for i in range(KG):
            gathers[i].wait()
            scatters.append(
                pltpu.async_copy(rows_v.at[i], acc_sh.at[idx_v.at[1, j0 + i]],
                                 sem_s, add=True))
        for d in scatters:
            d.wait()
        return carry

    lax.fori_loop(0, NG, body, 0)
    plsc.subcore_barrier()
    row0 = s * ROWS_PT
    pltpu.sync_copy(acc_sh.at[pl.ds(row0, ROWS_PT)],
                    out_hbm.at[c, pl.ds(row0, ROWS_PT)])


def kernel(x, edge_index, batch, W1, b1, W2, b2, Wc, bc):
    y_blk = pl.pallas_call(
        _mm1_body,
        out_shape=jax.ShapeDtypeStruct((YB, 128), jnp.float32),
    )(x.reshape(YB, 8 * DD), W1)
    y = y_blk.reshape(NN, HH)

    src_t = jnp.pad(edge_index[0], (0, EPAD - EE),
                    constant_values=NN).reshape(NT, CH, CHW)
    dst_t = jnp.pad(edge_index[1], (0, EPAD - EE),
                    constant_values=NN).reshape(NT, CH, CHW)

    partials = _sc_agg(y, src_t, dst_t)
    p_blk = partials.reshape(NC, PB, 128)

    out = pl.pallas_call(
        _final_body,
        out_shape=jax.ShapeDtypeStruct((BB, 2), jnp.float32),
    )(p_blk, y_blk, b1.reshape(1, HH), W2, b2.reshape(1, HH),
      batch.reshape(YB, 8).T, Wc, bc.reshape(1, 2))
    return out
